# Initial kernel scaffold; baseline (speedup 1.0000x reference)
import jax
import jax.numpy as jnp
from jax.experimental import pallas as pl


def kernel(value, depth, pos, params):
    raise NotImplementedError("write your pallas kernel here")


# speedup vs baseline: 2.1219x; 2.1219x over previous
import jax
import jax.numpy as jnp
from jax.experimental import pallas as pl

_EMBED = 1024
_HEADS = 16
_DHEAD = 64
_WINDOW = 64
_NCLUSTERS = 32
_ROWBLK = 512


def _embed_body(val_ref, dep_ref, p0_ref, p1_ref, p2_ref,
                tok_ref, depemb_ref, s0_ref, s1_ref, s2_ref,
                ov_ref, od_ref, o0_ref, o1_ref, o2_ref):
    def onehot(ids, n):
        io = jax.lax.broadcasted_iota(jnp.int32, (ids.shape[0], n), 1)
        return (ids[:, None] == io).astype(jnp.float32)

    hp = jax.lax.Precision.HIGHEST
    ov_ref[...] = jnp.dot(onehot(val_ref[...], 32), tok_ref[...], precision=hp,
                          preferred_element_type=jnp.float32)
    od_ref[...] = jnp.dot(onehot(dep_ref[...], 32), depemb_ref[...], precision=hp,
                          preferred_element_type=jnp.float32)
    for ids_ref, tab_ref, o_ref in ((p0_ref, s0_ref, o0_ref),
                                    (p1_ref, s1_ref, o1_ref),
                                    (p2_ref, s2_ref, o2_ref)):
        o_ref[...] = jnp.dot(onehot(ids_ref[...], 384), tab_ref[...], precision=hp,
                             preferred_element_type=jnp.float32)


def _embed(val, dep, p0, p1, p2, tok_p, dep_p, s0, s1, s2):
    n = val.shape[0]
    grid = n // _ROWBLK
    ids_spec = pl.BlockSpec((_ROWBLK,), lambda i: (i,))
    tab32 = pl.BlockSpec((32, _EMBED), lambda i: (0, 0))
    tab384 = pl.BlockSpec((384, _EMBED), lambda i: (0, 0))
    row = pl.BlockSpec((_ROWBLK, _EMBED), lambda i: (i, 0))
    out = jax.ShapeDtypeStruct((n, _EMBED), jnp.float32)
    return pl.pallas_call(
        _embed_body,
        grid=(grid,),
        in_specs=[ids_spec] * 5 + [tab32, tab32, tab384, tab384, tab384],
        out_specs=[row] * 5,
        out_shape=[out] * 5,
    )(val, dep, p0, p1, p2, tok_p, dep_p, s0, s1, s2)


def _qkv_body(h_ref, wq_ref, wk_ref, wv_ref, q_ref, k_ref, v_ref):
    h = h_ref[...]
    q_ref[...] = jnp.dot(h, wq_ref[...], preferred_element_type=jnp.float32)
    k_ref[...] = jnp.dot(h, wk_ref[...], preferred_element_type=jnp.float32)
    v_ref[...] = jnp.dot(h, wv_ref[...], preferred_element_type=jnp.float32)


def _qkv(h, wq, wk, wv):
    n = h.shape[0]
    grid = n // _ROWBLK
    row = pl.BlockSpec((_ROWBLK, _EMBED), lambda i: (i, 0))
    w = pl.BlockSpec((_EMBED, _EMBED), lambda i: (0, 0))
    out = jax.ShapeDtypeStruct((n, _EMBED), jnp.float32)
    return pl.pallas_call(
        _qkv_body,
        grid=(grid,),
        in_specs=[row, w, w, w],
        out_specs=[row, row, row],
        out_shape=[out, out, out],
    )(h, wq, wk, wv)


def _topk_idx(d, k):
    rows, t = d.shape
    iota = jax.lax.broadcasted_iota(jnp.int32, (rows, t), 1)
    cols = []
    for _ in range(k):
        m = jnp.max(d, axis=1, keepdims=True)
        eq = d >= m
        idx = jnp.min(jnp.where(eq, iota, t), axis=1)
        cols.append(idx[:, None])
        d = jnp.where(iota == idx[:, None], -jnp.inf, d)
    return jnp.concatenate(cols, axis=1)


def _split3(x):
    b1 = x.astype(jnp.bfloat16).astype(jnp.float32)
    r = x - b1
    b2 = r.astype(jnp.bfloat16).astype(jnp.float32)
    return b1, b2, r - b2


def _exact_gather(oh, x):
    b1, b2, b3 = _split3(x)
    acc = jnp.dot(oh, b1, preferred_element_type=jnp.float32)
    acc = acc + jnp.dot(oh, b2, preferred_element_type=jnp.float32)
    return acc + jnp.dot(oh, b3, preferred_element_type=jnp.float32)


def _sel_body(q_ref, k_ref, v_ref, dq_ref, dk_ref, km_ref,
              dm_ref, mask_ref, vg_ref, qi_ref, ki_ref):
    t = q_ref.shape[1]
    c, w, d = _NCLUSTERS, _WINDOW, _DHEAD
    q = q_ref[0]
    k = k_ref[0]
    v = v_ref[0]
    km = km_ref[0, 0]

    qi = _topk_idx(dq_ref[0], w)
    ki = _topk_idx(dk_ref[0], w)
    qi_ref[0] = qi
    ki_ref[0] = ki

    iota_t = jax.lax.broadcasted_iota(jnp.int32, (c, w, t), 2)
    oh_q = (qi[:, :, None] == iota_t).astype(jnp.float32).reshape(c * w, t)
    oh_k3 = (ki[:, :, None] == iota_t).astype(jnp.float32)
    oh_k = oh_k3.reshape(c * w, t)

    qg = _exact_gather(oh_q, q).reshape(c, w, d)
    kg = _exact_gather(oh_k, k).reshape(c, w, d)
    vg_ref[0] = _exact_gather(oh_k, v).reshape(c, w, d)

    dots = jax.lax.dot_general(qg, kg, (((2,), (2,)), ((0,), (0,))),
                               preferred_element_type=jnp.float32)
    dm_ref[0] = dots * (d ** -0.5)
    causal = ki[:, None, :] <= qi[:, :, None]
    kmg = jnp.sum(oh_k3 * km[None, None, :], axis=2)[:, None, :] > 0.5
    mask_ref[0] = (causal & kmg).astype(jnp.float32)


def _route_sel(q, k, v, dq, dk, km, num_heads):
    bh, t, d = q.shape
    c, w = _NCLUSTERS, _WINDOW
    qkv = pl.BlockSpec((1, t, d), lambda i: (i, 0, 0))
    dspec = pl.BlockSpec((1, c, t), lambda i: (i, 0, 0))
    kmspec = pl.BlockSpec((1, 1, t), lambda i: (i // num_heads, 0, 0))
    cw = pl.BlockSpec((1, c, w, w), lambda i: (i, 0, 0, 0))
    gspec = pl.BlockSpec((1, c, w, d), lambda i: (i, 0, 0, 0))
    ispec = pl.BlockSpec((1, c, w), lambda i: (i, 0, 0))
    return pl.pallas_call(
        _sel_body,
        grid=(bh,),
        in_specs=[qkv, qkv, qkv, dspec, dspec, kmspec],
        out_specs=[cw, cw, gspec, ispec, ispec],
        out_shape=[jax.ShapeDtypeStruct((bh, c, w, w), jnp.float32),
                   jax.ShapeDtypeStruct((bh, c, w, w), jnp.float32),
                   jax.ShapeDtypeStruct((bh, c, w, d), jnp.float32),
                   jax.ShapeDtypeStruct((bh, c, w), jnp.int32),
                   jax.ShapeDtypeStruct((bh, c, w), jnp.int32)],
    )(q, k, v, dq, dk, km)


def _out_body(og_ref, qi_ref, o_ref):
    t = o_ref.shape[1]
    c, w, d = _NCLUSTERS, _WINDOW, _DHEAD
    qi = qi_ref[0]
    og = og_ref[0]

    iota_t = jax.lax.broadcasted_iota(jnp.int32, (c, w, t), 2)
    oh_q = (qi[:, :, None] == iota_t).astype(jnp.float32).reshape(c * w, t)

    counts = jnp.sum(oh_q, axis=0)
    o_ref[0] = jnp.zeros((t, d), jnp.float32)
    iota_wt = jax.lax.broadcasted_iota(jnp.int32, (w, t), 1)

    def scatter_step(ci, _):
        selc = jax.lax.broadcasted_iota(jnp.int32, (c, 1), 0) == ci
        qi_c = jnp.sum(jnp.where(selc, qi, 0), axis=0)
        ohc = (qi_c[:, None] == iota_wt).astype(jnp.float32)
        og_c = jnp.sum(jnp.where(selc[:, :, None], og, 0.0), axis=0)
        o1, o2, o3 = _split3(og_c)
        contrib = jax.lax.dot_general(ohc, o1, (((0,), (0,)), ((), ())),
                                      preferred_element_type=jnp.float32)
        contrib = contrib + jax.lax.dot_general(
            ohc, o2, (((0,), (0,)), ((), ())),
            preferred_element_type=jnp.float32)
        contrib = contrib + jax.lax.dot_general(
            ohc, o3, (((0,), (0,)), ((), ())),
            preferred_element_type=jnp.float32)
        o_ref[0] = o_ref[0] + contrib
        return 0

    jax.lax.fori_loop(0, c, scatter_step, 0)
    o_ref[0] = o_ref[0] / jnp.maximum(counts, 1.0)[:, None]


def _route_out(og, qi, t):
    bh = og.shape[0]
    c, w, d = _NCLUSTERS, _WINDOW, _DHEAD
    gspec = pl.BlockSpec((1, c, w, d), lambda i: (i, 0, 0, 0))
    ispec = pl.BlockSpec((1, c, w), lambda i: (i, 0, 0))
    return pl.pallas_call(
        _out_body,
        grid=(bh,),
        in_specs=[gspec, ispec],
        out_specs=pl.BlockSpec((1, t, d), lambda i: (i, 0, 0)),
        out_shape=jax.ShapeDtypeStruct((bh, t, d), jnp.float32),
    )(og, qi)


def _wo_body(x_ref, o_ref, wo_ref, out_ref):
    out_ref[...] = x_ref[...] + jnp.dot(o_ref[...], wo_ref[...],
                                        preferred_element_type=jnp.float32)


def _wo_res(x, o, wo):
    n = x.shape[0]
    grid = n // _ROWBLK
    row = pl.BlockSpec((_ROWBLK, _EMBED), lambda i: (i, 0))
    return pl.pallas_call(
        _wo_body,
        grid=(grid,),
        in_specs=[row, row, pl.BlockSpec((_EMBED, _EMBED), lambda i: (0, 0))],
        out_specs=row,
        out_shape=jax.ShapeDtypeStruct((n, _EMBED), jnp.float32),
    )(x, o, wo)


def _ff_body(x_ref, h_ref, w1_ref, b1_ref, w2_ref, b2_ref, out_ref):
    f = jax.nn.gelu(jnp.dot(h_ref[...], w1_ref[...],
                            preferred_element_type=jnp.float32) + b1_ref[...])
    f = jnp.dot(f, w2_ref[...], preferred_element_type=jnp.float32) + b2_ref[...]
    out_ref[...] = x_ref[...] + f


def _ff(x, h, w1, b1, w2, b2):
    n = x.shape[0]
    blk = 256
    grid = n // blk
    hid = w1.shape[1]
    row = pl.BlockSpec((blk, _EMBED), lambda i: (i, 0))
    return pl.pallas_call(
        _ff_body,
        grid=(grid,),
        in_specs=[row, row,
                  pl.BlockSpec((_EMBED, hid), lambda i: (0, 0)),
                  pl.BlockSpec((hid,), lambda i: (0,)),
                  pl.BlockSpec((hid, _EMBED), lambda i: (0, 0)),
                  pl.BlockSpec((_EMBED,), lambda i: (0,))],
        out_specs=row,
        out_shape=jax.ShapeDtypeStruct((n, _EMBED), jnp.float32),
    )(x, h, w1, b1, w2, b2)


def _head_body(x_ref, w_ref, out_ref):
    out_ref[...] = jnp.dot(x_ref[...], w_ref[...],
                           preferred_element_type=jnp.float32)


def _head(x, w_pad):
    n = x.shape[0]
    grid = n // _ROWBLK
    vpad = w_pad.shape[1]
    return pl.pallas_call(
        _head_body,
        grid=(grid,),
        in_specs=[pl.BlockSpec((_ROWBLK, _EMBED), lambda i: (i, 0)),
                  pl.BlockSpec((_EMBED, vpad), lambda i: (0, 0))],
        out_specs=pl.BlockSpec((_ROWBLK, vpad), lambda i: (i, 0)),
        out_shape=jax.ShapeDtypeStruct((n, vpad), jnp.float32),
    )(x, w_pad)


def _ln(x, g, b):
    mu = jnp.mean(x, axis=-1, keepdims=True)
    var = jnp.var(x, axis=-1, keepdims=True)
    return (x - mu) / jnp.sqrt(var + 1e-5) * g + b


def _l2n(x):
    return x / jnp.maximum(jnp.linalg.norm(x, axis=-1, keepdims=True), 1e-6)


def kernel(value, depth, pos, params):
    b, t = value.shape
    n = b * t
    nv = params['head'].shape[1]

    value = value.astype(jnp.int32)
    depth = depth.astype(jnp.int32)
    pos = pos.astype(jnp.int32)

    def pad_rows(tab, rows):
        return jnp.pad(tab, ((0, rows - tab.shape[0]), (0, 0)))

    tok_p = pad_rows(params['tok'], 32)
    dep_p = pad_rows(params['dep'], 32)
    spa = params['spa']
    s0, s1, s2 = (pad_rows(spa[a], 384) for a in range(3))

    posf = pos.reshape(n, 3)
    tok_g, dep_g, s0g, s1g, s2g = _embed(value.reshape(n), depth.reshape(n),
                                         posf[:, 0], posf[:, 1], posf[:, 2],
                                         tok_p, dep_p, s0, s1, s2)
    x = (tok_g + dep_g).reshape(b, t, _EMBED)
    for sg in (s0g, s1g, s2g):
        x = x + sg.reshape(b, t, _EMBED)
    sos = jnp.broadcast_to(params['sos'], (b, 1, _EMBED))
    x = jnp.concatenate([sos, x[:, :-1, :]], axis=1)

    km = (value != 0).astype(jnp.float32).reshape(b, 1, t)

    for p in params['layers']:
        hx = _ln(x, p['ln1_g'], p['ln1_b'])

        def split(w):
            return (hx @ w).reshape(b, t, _HEADS, _DHEAD).transpose(0, 2, 1, 3)

        qh, kh, vh = split(p['wq']), split(p['wk']), split(p['wv'])
        qnh, knh = _l2n(qh), _l2n(kh)
        mn = _l2n(p['means'])
        dq = jnp.einsum('bhtd,hcd->bhct', qnh, mn)
        dk = jnp.einsum('bhtd,hcd->bhct', knh, mn)

        def flat(z):
            return z.reshape(b * _HEADS, t, _DHEAD)

        dm, maskf, vg, qi, ki = _route_sel(flat(qh), flat(kh), flat(vh),
                                           dq.reshape(b * _HEADS, _NCLUSTERS, t),
                                           dk.reshape(b * _HEADS, _NCLUSTERS, t),
                                           km, _HEADS)
        shape5 = (b, _HEADS, _NCLUSTERS, _WINDOW, _WINDOW)
        mask_b = maskf.reshape(shape5) > 0.5
        dmask = jnp.where(mask_b, dm.reshape(shape5), -1e9)
        attn = jax.nn.softmax(dmask, axis=-1) * mask_b
        og = jnp.einsum('bhcij,bhcjd->bhcid', attn,
                        vg.reshape(b, _HEADS, _NCLUSTERS, _WINDOW, _DHEAD))
        o = _route_out(og.reshape(b * _HEADS, _NCLUSTERS, _WINDOW, _DHEAD),
                       qi, t)
        o = (o.reshape(b, _HEADS, t, _DHEAD)
             .transpose(0, 2, 1, 3)
             .reshape(b, t, _HEADS * _DHEAD)) @ p['wo']
        x = x + o
        h2 = _ln(x, p['ln2_g'], p['ln2_b'])
        f = jax.nn.gelu(h2 @ p['ff1'] + p['ff1_b']) @ p['ff2'] + p['ff2_b']
        x = x + f

    head_pad = jnp.pad(params['head'], ((0, 0), (0, 128 - nv)))
    logits = _head(x.reshape(n, _EMBED), head_pad)[:, :nv]
    return logits.reshape(b, t, nv)
